# baseline (speedup 1.0000x reference)
import functools

import jax
import jax.numpy as jnp
from jax.experimental import pallas as pl

_LAYERS = [(32, 4, 32, True), (128, 4, 32, True), (128, 1, 32, False)]


def _mm_body(x_ref, w_ref, o_ref):
    o_ref[...] = jnp.dot(x_ref[...], w_ref[...], preferred_element_type=jnp.float32)


def _mm(x, w, bn):
    n, k = x.shape
    m = w.shape[1]
    assert n % bn == 0, (n, bn)
    return pl.pallas_call(
        _mm_body,
        grid=(n // bn,),
        in_specs=[
            pl.BlockSpec((bn, k), lambda i: (i, 0)),
            pl.BlockSpec((k, m), lambda i: (0, 0)),
        ],
        out_specs=pl.BlockSpec((bn, m), lambda i: (i, 0)),
        out_shape=jax.ShapeDtypeStruct((n, m), jnp.float32),
    )(x, w)


def _segment_softmax(alpha, seg, num_segments):
    m = jax.ops.segment_max(alpha, seg, num_segments)
    m = jnp.where(jnp.isfinite(m), m, 0.0)
    ex = jnp.exp(alpha - m[seg])
    denom = jax.ops.segment_sum(ex, seg, num_segments)
    return ex / (denom[seg] + 1e-16)


def _gat_layer(x, src, dst, a_edge, p, heads, out_ch, concat):
    num_nodes = x.shape[0]
    h = _mm(x, p['W'], 2000).reshape(num_nodes, heads, out_ch)
    a_src = (h * p['att_src']).sum(-1)
    a_dst = (h * p['att_dst']).sum(-1)
    alpha = a_src[src] + a_dst[dst] + a_edge
    alpha = jax.nn.leaky_relu(alpha, 0.2)
    alpha = _segment_softmax(alpha, dst, num_nodes)
    msg = h[src] * alpha[..., None]
    out = jax.ops.segment_sum(msg, dst, num_nodes)
    if concat:
        out = out.reshape(num_nodes, heads * out_ch)
    else:
        out = out.mean(axis=1)
    return out + p['bias']


def kernel(x, edge_index, edge_attr, params):
    src = edge_index[0].astype(jnp.int32)
    dst = edge_index[1].astype(jnp.int32)
    eid = jnp.arange(src.shape[0], dtype=jnp.int32)
    dst, src, eid = jax.lax.sort((dst, src, eid), num_keys=1)
    we_atts = []
    for p in params:
        h_, c_ = p['att_src'].shape[1], p['att_src'].shape[2]
        we = p['W_e'].reshape(p['W_e'].shape[0], h_, c_)
        we_atts.append((we * p['att_edge']).sum(-1))
    we_all = jnp.concatenate(we_atts, axis=1)
    we_all = jnp.pad(we_all, ((0, 0), (0, 16 - we_all.shape[1])))
    a_edge_all = _mm(edge_attr, we_all, 16000)[eid]

    h = x
    off = 0
    n_layers = len(params)
    for i, p in enumerate(params):
        heads, out_ch = p['att_src'].shape[1], p['att_src'].shape[2]
        concat = i < n_layers - 1 or p['bias'].shape[0] == heads * out_ch
        a_edge = a_edge_all[:, off:off + heads]
        off += heads
        h = _gat_layer(h, src, dst, a_edge, p, heads, out_ch,
                       concat=(p['bias'].shape[0] == heads * out_ch))
        if i < n_layers - 1:
            h = jax.nn.elu(h)
    return h


# speedup vs baseline: 40.0511x; 40.0511x over previous
import functools

import jax
import jax.numpy as jnp
from jax import lax
from jax.experimental import pallas as pl
from jax.experimental.pallas import tpu as pltpu
from jax.experimental.pallas import tpu_sc as plsc

N = 50000
E = 800000
NW = 32
EPW = E // NW
KA = 1000
NGA = KA // 16
TAIL = KA - NGA * 16
NBLK_AB = EPW // KA
NR = 416
P = 128
N_PAD = NR * P
MAGIC, MSHIFT = 20165, 23
CH = 16
RECSZ = E + NW * P * (CH - 1) + 128
LTRASH = EPW + P * (CH - 1) + 24
LSZ = LTRASH + 224
SENT = 511 << 16
KC = 128
OFFS = (0, 4, 8, 12)

_MESH = dict(core_axis_name="c", subcore_axis_name="s")


def _al8(x):
    return pl.multiple_of(x, 8)

def _iota16():
    return lax.broadcasted_iota(jnp.int32, (16,), 0)


def _wid():
    return lax.axis_index("s") * 2 + lax.axis_index("c")


def _k0_aedge(edge_attr, we_all):
    be = 16000

    def body(ea_ref, w_ref, out_ref, mx_ref):
        i = pl.program_id(0)
        r = jnp.dot(ea_ref[...], w_ref[...], preferred_element_type=jnp.float32)
        out_ref[...] = r
        m = jnp.max(r, axis=0, keepdims=True)

        @pl.when(i == 0)
        def _():
            mx_ref[...] = m

        @pl.when(i > 0)
        def _():
            mx_ref[...] = jnp.maximum(mx_ref[...], m)

    return pl.pallas_call(
        body,
        grid=(E // be,),
        in_specs=[pl.BlockSpec((be, 16), lambda i: (i, 0)),
                  pl.BlockSpec((16, 16), lambda i: (0, 0))],
        out_specs=[pl.BlockSpec((be, 16), lambda i: (i, 0)),
                   pl.BlockSpec((1, 16), lambda i: (0, 0))],
        out_shape=[jax.ShapeDtypeStruct((E, 16), jnp.float32),
                   jax.ShapeDtypeStruct((1, 16), jnp.float32)],
    )(edge_attr, we_all)


def _k1_dense(x, w_pad, msrc, mdst):
    bn = 2000
    din = x.shape[1]

    def body(x_ref, w_ref, ms_ref, md_ref, hx_ref, ad_ref, sm_ref):
        i = pl.program_id(0)
        h = jnp.dot(x_ref[...], w_ref[...], preferred_element_type=jnp.float32)
        asrc = jnp.dot(h, ms_ref[...], preferred_element_type=jnp.float32)
        adst = jnp.dot(h, md_ref[...], preferred_element_type=jnp.float32)
        hx_ref[...] = jnp.concatenate([h, asrc], axis=1)
        ad_ref[...] = adst
        m = jnp.max(asrc, axis=0, keepdims=True)

        @pl.when(i == 0)
        def _():
            sm_ref[...] = m

        @pl.when(i > 0)
        def _():
            sm_ref[...] = jnp.maximum(sm_ref[...], m)

    return pl.pallas_call(
        body,
        grid=(N // bn,),
        in_specs=[pl.BlockSpec((bn, din), lambda i: (i, 0)),
                  pl.BlockSpec((din, 128), lambda i: (0, 0)),
                  pl.BlockSpec((128, 16), lambda i: (0, 0)),
                  pl.BlockSpec((128, 16), lambda i: (0, 0))],
        out_specs=[pl.BlockSpec((bn, 144), lambda i: (i, 0)),
                   pl.BlockSpec((bn, 16), lambda i: (i, 0)),
                   pl.BlockSpec((1, 16), lambda i: (0, 0))],
        out_shape=[jax.ShapeDtypeStruct((N, 144), jnp.float32),
                   jax.ShapeDtypeStruct((N_PAD, 16), jnp.float32),
                   jax.ShapeDtypeStruct((1, 16), jnp.float32)],
    )(x, w_pad, msrc, mdst)


def _k2_tdstb(adst16, smax4, emax4):
    bn = 2048

    def body(ad_ref, sm_ref, em_ref, t_ref):
        a4 = ad_ref[...][:, :4]
        z = a4 + sm_ref[...] + em_ref[...]
        b = jnp.maximum(z, 0.2 * z)
        t_ref[...] = jnp.concatenate([a4, b], axis=1)

    return pl.pallas_call(
        body,
        grid=(N_PAD // bn,),
        in_specs=[pl.BlockSpec((bn, 16), lambda i: (i, 0)),
                  pl.BlockSpec((1, 4), lambda i: (0, 0)),
                  pl.BlockSpec((1, 4), lambda i: (0, 0))],
        out_specs=pl.BlockSpec((bn, 8), lambda i: (i, 0)),
        out_shape=jax.ShapeDtypeStruct((N_PAD, 8), jnp.float32),
    )(adst16, smax4, emax4)


def _k3_norm(acc, bias_pad, expm, hc_out, elu):
    bn = 2000

    def body(a_ref, b_ref, e_ref, o_ref):
        a = a_ref[...]
        recip = 1.0 / (a[:, 128:144] + 1e-16)
        scale = jnp.dot(recip, e_ref[...], preferred_element_type=jnp.float32)
        o = a[:, 0:128] * scale + b_ref[...]
        if elu:
            o = jnp.where(o > 0.0, o, jnp.exp(o) - 1.0)
        o_ref[...] = o[:, :hc_out]

    return pl.pallas_call(
        body,
        grid=(N // bn,),
        in_specs=[pl.BlockSpec((bn, 144), lambda i: (i, 0)),
                  pl.BlockSpec((1, 128), lambda i: (0, 0)),
                  pl.BlockSpec((16, 128), lambda i: (0, 0))],
        out_specs=pl.BlockSpec((bn, hc_out), lambda i: (i, 0)),
        out_shape=jax.ShapeDtypeStruct((N, hc_out), jnp.float32),
    )(acc, bias_pad, expm)


def _pass_a(dst):
    @functools.partial(
        pl.kernel,
        out_type=jax.ShapeDtypeStruct((NW, P * 16), jnp.int32),
        mesh=plsc.VectorSubcoreMesh(**_MESH),
        compiler_params=pltpu.CompilerParams(needs_layout_passes=False, use_tc_tiling_on_sc=False),
        scratch_types=[pltpu.VMEM((1008,), jnp.int32),
                       pltpu.VMEM((P * 16,), jnp.int32)],
    )
    def k(dst_hbm, cnt_hbm, dbuf, cbuf):
        wid = _wid()
        lane = _iota16()
        ones = jnp.ones((16,), jnp.int32)

        def zero(i, _):
            cbuf[pl.ds(_al8(i * 16), 16)] = jnp.zeros((16,), jnp.int32)
            return 0
        lax.fori_loop(0, P, zero, 0)

        def blk(ib, _):
            pltpu.sync_copy(dst_hbm.at[pl.ds(_al8(wid * EPW + ib * KA), KA)],
                            dbuf.at[pl.ds(0, KA)])

            def grp(g, _):
                d = jnp.clip(dbuf[pl.ds(_al8(g * 16), 16)], 0, N_PAD - 1)
                b = (d * MAGIC) >> MSHIFT
                plsc.addupdate_scatter(cbuf, [b * 16 + lane], ones)
                return 0
            lax.fori_loop(0, NGA, grp, 0)
            d = jnp.clip(dbuf[pl.ds(NGA * 16, 16)], 0, N_PAD - 1)
            b = (d * MAGIC) >> MSHIFT
            plsc.addupdate_scatter(cbuf, [b * 16 + lane], ones,
                                   mask=lane < TAIL)
            return 0
        lax.fori_loop(0, NBLK_AB, blk, 0)
        pltpu.sync_copy(cbuf, cnt_hbm.at[wid])

    return k(dst)


def _pass_b(src, dst, curinit, lloffs, fills, padls, nchs, gstarts):
    @functools.partial(
        pl.kernel,
        out_type=[jax.ShapeDtypeStruct((RECSZ,), jnp.int32),
                  jax.ShapeDtypeStruct((RECSZ,), jnp.int32)],
        mesh=plsc.VectorSubcoreMesh(**_MESH),
        compiler_params=pltpu.CompilerParams(needs_layout_passes=False, use_tc_tiling_on_sc=False),
        scratch_types=[pltpu.VMEM((1008,), jnp.int32),
                       pltpu.VMEM((1008,), jnp.int32),
                       pltpu.VMEM((144,), jnp.int32),
                       pltpu.VMEM((16,), jnp.int32),
                       pltpu.VMEM((16,), jnp.int32),
                       pltpu.VMEM((LSZ,), jnp.int32),
                       pltpu.VMEM((LSZ,), jnp.int32),
                       pltpu.VMEM((144,), jnp.int32),
                       pltpu.VMEM((144,), jnp.int32),
                       pltpu.VMEM((144,), jnp.int32),
                       pltpu.VMEM((144,), jnp.int32),
                       pltpu.VMEM((144,), jnp.int32),
                       pltpu.VMEM((16,), jnp.int32),
                       pltpu.SemaphoreType.DMA],
    )
    def k(src_hbm, dst_hbm, cur_hbm, ll_hbm, fi_hbm, pa_hbm, nc_hbm, gs_hbm,
          ra_hbm, rb_hbm,
          sbuf, dbuf, curb, tmp, postmp, lsd, lei, lb, fb, pb, nb, gb,
          dummy, sem):
        wid = _wid()
        lane = _iota16()
        sentv = jnp.full((16,), SENT, jnp.int32)
        zerov = jnp.zeros((16,), jnp.int32)
        pltpu.sync_copy(cur_hbm.at[wid], curb)

        def group(o, base, valid_n):
            s = jnp.clip(sbuf[pl.ds(_al8(o), 16)], 0, N - 1)
            d = jnp.clip(dbuf[pl.ds(_al8(o), 16)], 0, N_PAD - 1)
            b = (d * MAGIC) >> MSHIFT
            if valid_n is not None:
                b = jnp.where(lane < valid_n, b, P)
            sb, lid = plsc.sort_key_val(b, lane)
            tmp[pl.ds(0, 16)] = sb
            prev = plsc.load_gather(tmp, [jnp.maximum(lane - 1, 0)])
            startf = jnp.logical_or(lane == 0, sb != prev)
            run_start = plsc.cummax(jnp.where(startf, lane, 0))
            rank = lane - run_start
            basep = plsc.load_gather(curb, [sb])
            pos_s = basep + rank
            nxt = plsc.load_gather(tmp, [jnp.minimum(lane + 1, 15)])
            endf = jnp.logical_or(lane == 15, sb != nxt)
            plsc.store_scatter(curb, [sb], pos_s + 1, mask=endf)
            plsc.store_scatter(postmp, [lid], pos_s)
            pos = postmp[pl.ds(0, 16)]
            pk = s | ((d - b * NR) << 16)
            if valid_n is not None:
                pk = jnp.where(lane < valid_n, pk, SENT)
            plsc.store_scatter(lsd, [pos], pk)
            plsc.store_scatter(lei, [pos], base + o + lane)

        def blk(ib, _):
            base = wid * EPW + ib * KA
            pltpu.sync_copy(src_hbm.at[pl.ds(_al8(base), KA)], sbuf.at[pl.ds(0, KA)])
            pltpu.sync_copy(dst_hbm.at[pl.ds(_al8(base), KA)], dbuf.at[pl.ds(0, KA)])

            def grp(g, _):
                group(g * 16, base, None)
                return 0
            lax.fori_loop(0, NGA, grp, 0)
            group(NGA * 16, base, TAIL)
            return 0
        lax.fori_loop(0, NBLK_AB, blk, 0)

        pltpu.sync_copy(fi_hbm.at[wid], fb)
        pltpu.sync_copy(pa_hbm.at[wid], pb)
        for bb in range(8):
            fv = fb[pl.ds(bb * 16, 16)]
            pv = pb[pl.ds(bb * 16, 16)]
            for r in range(16):
                idx = fv[r] + lane
                m = lane < pv[r]
                plsc.store_scatter(lsd, [idx], sentv, mask=m)
                plsc.store_scatter(lei, [idx], zerov, mask=m)

        pltpu.sync_copy(ll_hbm.at[wid], lb)
        pltpu.sync_copy(nc_hbm.at[wid], nb)
        pltpu.sync_copy(gs_hbm.at[wid], gb)
        prev_n = None

        def drain(count):
            def dr(i, _):
                pltpu.make_async_copy(
                    ra_hbm.at[pl.ds(0, CH)], dummy, sem).wait()
                return 0
            lax.fori_loop(0, count, dr, 0)

        for bb in range(8):
            lv = lb[pl.ds(bb * 16, 16)]
            nv = nb[pl.ds(bb * 16, 16)]
            gv = gb[pl.ds(bb * 16, 16)]
            for r in range(16):
                l0, n0, g0 = lv[r], nv[r], gv[r]

                def cp(ic, _):
                    pltpu.async_copy(lsd.at[pl.ds(_al8(l0 + ic * CH), CH)],
                                     ra_hbm.at[pl.ds(_al8(g0 + ic * CH), CH)], sem)
                    pltpu.async_copy(lei.at[pl.ds(_al8(l0 + ic * CH), CH)],
                                     rb_hbm.at[pl.ds(_al8(g0 + ic * CH), CH)], sem)
                    return 0
                lax.fori_loop(0, n0, cp, 0)
                if prev_n is not None:
                    drain(2 * prev_n)
                prev_n = n0
        drain(2 * prev_n)

    return k(src, dst, curinit, lloffs, fills, padls, nchs, gstarts)


def _pass_c(heads, off, nch, reca, recb_, aedge, hext, tdstb, bins):
    @functools.partial(
        pl.kernel,
        out_type=jax.ShapeDtypeStruct((N_PAD, 144), jnp.float32),
        mesh=plsc.VectorSubcoreMesh(**_MESH),
        compiler_params=pltpu.CompilerParams(needs_layout_passes=False, use_tc_tiling_on_sc=False),
        scratch_types=[pltpu.VMEM((NR, 144), jnp.float32),
                       pltpu.VMEM((NR, 8), jnp.float32),
                       pltpu.VMEM((16,), jnp.int32),
                       pltpu.VMEM((KC,), jnp.int32),
                       pltpu.VMEM((KC,), jnp.int32),
                       pltpu.VMEM((KC, 16), jnp.float32),
                       pltpu.VMEM((KC, 144), jnp.float32),
                       pltpu.VMEM((KC,), jnp.int32),
                       pltpu.VMEM((KC,), jnp.int32),
                       pltpu.VMEM((KC,), jnp.int32),
                       pltpu.VMEM((KC, 16), jnp.float32),
                       pltpu.SemaphoreType.DMA,
                       pltpu.SemaphoreType.DMA],
    )
    def k(ra_hbm, rb_hbm, ae_hbm, hx_hbm, td_hbm, bins_hbm, out_hbm,
          acc, tl, binb, ra, rb, aeb, hbuf, sidx, eidx, dbuf, wbuf,
          sem_a, sem_h):
        wid = _wid()
        lane = _iota16()
        z16 = jnp.zeros((16,), jnp.int32)
        zf = jnp.zeros((16,), jnp.float32)
        mask_h = jnp.where(lane < heads, 1.0, 0.0).astype(jnp.float32)

        for j in range(4):
            part = wid * 4 + j
            node_base = part * NR
            pltpu.sync_copy(td_hbm.at[pl.ds(node_base, NR)], tl)
            pltpu.sync_copy(bins_hbm.at[part], binb)

            def za(i, _):
                for c in range(9):
                    acc[i, pl.ds(c * 16, 16)] = zf
                return 0
            lax.fori_loop(0, NR, za, 0)

            bv = binb[pl.ds(0, 16)]
            e0 = bv[0]
            ne = bv[1]
            nblk = (ne + KC - 1) >> 7

            def cblk(ib, _):
                eb = e0 + ib * KC
                pltpu.sync_copy(ra_hbm.at[pl.ds(_al8(eb), KC)], ra)
                pltpu.sync_copy(rb_hbm.at[pl.ds(_al8(eb), KC)], rb)
                rem = ne - ib * KC
                for g in range(KC // 16):
                    v = ra[pl.ds(g * 16, 16)]
                    ev = rb[pl.ds(g * 16, 16)]
                    sidx[pl.ds(g * 16, 16)] = jnp.clip(v & 0xFFFF, 0, N - 1)
                    eidx[pl.ds(g * 16, 16)] = jnp.clip(ev, 0, E - 1)
                    dbuf[pl.ds(g * 16, 16)] = (v >> 16) & 0x1FF
                cp_a = pltpu.async_copy(ae_hbm.at[eidx], aeb, sem_a)
                cp_h = pltpu.async_copy(hx_hbm.at[sidx], hbuf, sem_h)
                cp_a.wait()
                cp_h.wait()
                for g in range(KC // 16):
                    e16 = g * 16 + lane
                    dvr = dbuf[pl.ds(_al8(g * 16), 16)]
                    dv = jnp.minimum(dvr, NR - 1)
                    vm = jnp.logical_and(e16 < rem, dvr < NR)
                    for h in range(heads):
                        a_s = plsc.load_gather(hbuf, [e16, z16 + (128 + h)])
                        a_d = plsc.load_gather(tl, [dv, z16 + h])
                        bnd = plsc.load_gather(tl, [dv, z16 + (4 + h)])
                        a_e = plsc.load_gather(aeb, [e16, z16 + (off + h)])
                        al = a_s + a_d + a_e
                        al = jnp.maximum(al, 0.2 * al)
                        w = jnp.exp(al - bnd)
                        w = jnp.where(vm, w, 0.0)
                        plsc.store_scatter(wbuf, [e16, z16 + h], w)

                def acc_grp(g, _):
                    base16 = g * 16
                    dv16 = jnp.minimum(dbuf[pl.ds(_al8(base16), 16)], NR - 1)
                    for k in range(16):
                        i_row = base16 + k
                        d = dv16[k]
                        wrow = wbuf[i_row, pl.ds(0, 16)]
                        for c in range(nch):
                            hv = hbuf[i_row, pl.ds(c * 16, 16)]
                            plsc.addupdate(acc.at[d, pl.ds(c * 16, 16)],
                                           wrow[c // 2] * hv)
                        plsc.addupdate(acc.at[d, pl.ds(128, 16)], wrow * mask_h)
                    return 0
                lax.fori_loop(0, KC // 16, acc_grp, 0)
                return 0
            lax.fori_loop(0, nblk, cblk, 0)
            pltpu.sync_copy(acc, out_hbm.at[pl.ds(node_base, NR)])

    return k(reca, recb_, aedge, hext, tdstb, bins)


def kernel(x, edge_index, edge_attr, params):
    src = edge_index[0].astype(jnp.int32)
    dst = edge_index[1].astype(jnp.int32)
    f32 = jnp.float32
    i32 = jnp.int32

    we_cols, w_pads, msrcs, mdsts, biases, expms, hcs = [], [], [], [], [], [], []
    for p in params:
        heads = p['att_src'].shape[1]
        ch = p['att_src'].shape[2]
        hc = heads * ch
        din = p['W'].shape[0]
        we = p['W_e'].reshape(p['W_e'].shape[0], heads, ch)
        we_cols.append((we * p['att_edge']).sum(-1))
        w_pads.append(jnp.pad(p['W'], ((0, 0), (0, 128 - hc))))
        rows = jnp.arange(hc)
        msrc = jnp.zeros((128, 16), f32).at[rows, rows // ch].set(
            p['att_src'].reshape(hc))
        mdst = jnp.zeros((128, 16), f32).at[rows, rows // ch].set(
            p['att_dst'].reshape(hc))
        msrcs.append(msrc)
        mdsts.append(mdst)
        biases.append(jnp.pad(p['bias'], (0, 128 - p['bias'].shape[0]))
                      .reshape(1, 128))
        expms.append(jnp.zeros((16, 128), f32).at[rows // ch, rows].set(1.0))
        hcs.append(hc)

    we_all = jnp.concatenate(we_cols, axis=1)
    we_all = jnp.pad(we_all, ((0, 0), (0, 16 - we_all.shape[1])))
    aedge_all, emax16 = _k0_aedge(edge_attr, we_all)

    cnt = _pass_a(dst).reshape(NW, P, 16).sum(-1)
    rnd = ((cnt + CH - 1) // CH) * CH
    ne_pad = rnd.sum(0)
    astart = jnp.concatenate(
        [jnp.zeros((1,), i32), jnp.cumsum(ne_pad)])[:P]
    gstart = astart[None, :] + jnp.cumsum(rnd, axis=0) - rnd
    lloff = jnp.cumsum(rnd, axis=1) - rnd

    def _pad144(a, trail=0):
        ext = jnp.full((NW, 144 - P), trail, i32)
        return jnp.concatenate([a.astype(i32), ext], axis=1)

    curinit = _pad144(lloff).at[:, P].set(LTRASH)
    lloffs = _pad144(lloff)
    fills = _pad144(lloff + cnt)
    padls = _pad144(rnd - cnt)
    nchs = _pad144(rnd // CH)
    gstarts = _pad144(gstart)
    bins = jnp.concatenate(
        [astart[:, None], ne_pad[:, None], jnp.zeros((P, 14), i32)], axis=1)
    reca, recb = _pass_b(src, dst, curinit, lloffs, fills, padls, nchs,
                         gstarts)

    h = x
    layer_heads = [p['att_src'].shape[1] for p in params]
    for li, p in enumerate(params):
        heads = layer_heads[li]
        hext, adst16, smax = _k1_dense(h, w_pads[li], msrcs[li], mdsts[li])
        td = _k2_tdstb(adst16, smax[:, :4], emax16[:, OFFS[li]:OFFS[li] + 4])
        acc = _pass_c(heads, OFFS[li], (heads * 32) // 16,
                      reca, recb, aedge_all, hext, td, bins)
        out_w = hcs[li] if li < len(params) - 1 else params[li]['bias'].shape[0]
        h = _k3_norm(acc, biases[li], expms[li], out_w, elu=li < len(params) - 1)
    return h


# speedup vs baseline: 47.0025x; 1.1736x over previous
import functools

import jax
import jax.numpy as jnp
from jax import lax
from jax.experimental import pallas as pl
from jax.experimental.pallas import tpu as pltpu
from jax.experimental.pallas import tpu_sc as plsc

N = 50000
E = 800000
NW = 32
EPW = E // NW
KA = 1000
NGA = KA // 16
TAIL = KA - NGA * 16
NBLK_AB = EPW // KA
NR = 416
P = 128
N_PAD = NR * P
MAGIC, MSHIFT = 20165, 23
CH = 16
RECSZ = E + NW * P * (CH - 1) + 128
LTRASH = EPW + P * (CH - 1) + 24
LSZ = LTRASH + 224
SENT = 511 << 16
KC = 128
OFFS = (0, 4, 8, 12)

_MESH = dict(core_axis_name="c", subcore_axis_name="s")


def _al8(x):
    return pl.multiple_of(x, 8)

def _iota16():
    return lax.broadcasted_iota(jnp.int32, (16,), 0)


def _wid():
    return lax.axis_index("s") * 2 + lax.axis_index("c")


def _k0_aedge(edge_attr, we_all):
    be = 16000

    def body(ea_ref, w_ref, out_ref, mx_ref):
        i = pl.program_id(0)
        r = jnp.dot(ea_ref[...], w_ref[...], preferred_element_type=jnp.float32)
        out_ref[...] = r
        m = jnp.max(r, axis=0, keepdims=True)

        @pl.when(i == 0)
        def _():
            mx_ref[...] = m

        @pl.when(i > 0)
        def _():
            mx_ref[...] = jnp.maximum(mx_ref[...], m)

    return pl.pallas_call(
        body,
        grid=(E // be,),
        in_specs=[pl.BlockSpec((be, 16), lambda i: (i, 0)),
                  pl.BlockSpec((16, 16), lambda i: (0, 0))],
        out_specs=[pl.BlockSpec((be, 16), lambda i: (i, 0)),
                   pl.BlockSpec((1, 16), lambda i: (0, 0))],
        out_shape=[jax.ShapeDtypeStruct((E, 16), jnp.float32),
                   jax.ShapeDtypeStruct((1, 16), jnp.float32)],
    )(edge_attr, we_all)


def _k1_dense(x, w_pad, msrc, mdst):
    bn = 2000
    din = x.shape[1]

    def body(x_ref, w_ref, ms_ref, md_ref, hx_ref, ad_ref, sm_ref):
        i = pl.program_id(0)
        h = jnp.dot(x_ref[...], w_ref[...], preferred_element_type=jnp.float32)
        asrc = jnp.dot(h, ms_ref[...], preferred_element_type=jnp.float32)
        adst = jnp.dot(h, md_ref[...], preferred_element_type=jnp.float32)
        hx_ref[...] = jnp.concatenate([h, asrc], axis=1)
        ad_ref[...] = adst
        m = jnp.max(asrc, axis=0, keepdims=True)

        @pl.when(i == 0)
        def _():
            sm_ref[...] = m

        @pl.when(i > 0)
        def _():
            sm_ref[...] = jnp.maximum(sm_ref[...], m)

    return pl.pallas_call(
        body,
        grid=(N // bn,),
        in_specs=[pl.BlockSpec((bn, din), lambda i: (i, 0)),
                  pl.BlockSpec((din, 128), lambda i: (0, 0)),
                  pl.BlockSpec((128, 16), lambda i: (0, 0)),
                  pl.BlockSpec((128, 16), lambda i: (0, 0))],
        out_specs=[pl.BlockSpec((bn, 144), lambda i: (i, 0)),
                   pl.BlockSpec((bn, 16), lambda i: (i, 0)),
                   pl.BlockSpec((1, 16), lambda i: (0, 0))],
        out_shape=[jax.ShapeDtypeStruct((N, 144), jnp.float32),
                   jax.ShapeDtypeStruct((N_PAD, 16), jnp.float32),
                   jax.ShapeDtypeStruct((1, 16), jnp.float32)],
    )(x, w_pad, msrc, mdst)


def _k2_tdstb(adst16, smax4, emax4):
    bn = 2048

    def body(ad_ref, sm_ref, em_ref, t_ref):
        a4 = ad_ref[...][:, :4]
        z = a4 + sm_ref[...] + em_ref[...]
        b = jnp.maximum(z, 0.2 * z)
        t_ref[...] = jnp.concatenate([a4, b], axis=1)

    return pl.pallas_call(
        body,
        grid=(N_PAD // bn,),
        in_specs=[pl.BlockSpec((bn, 16), lambda i: (i, 0)),
                  pl.BlockSpec((1, 4), lambda i: (0, 0)),
                  pl.BlockSpec((1, 4), lambda i: (0, 0))],
        out_specs=pl.BlockSpec((bn, 8), lambda i: (i, 0)),
        out_shape=jax.ShapeDtypeStruct((N_PAD, 8), jnp.float32),
    )(adst16, smax4, emax4)


def _k3_norm(acc, bias_pad, expm, hc_out, elu):
    bn = 2000

    def body(a_ref, b_ref, e_ref, o_ref):
        a = a_ref[...]
        recip = 1.0 / (a[:, 128:144] + 1e-16)
        scale = jnp.dot(recip, e_ref[...], preferred_element_type=jnp.float32)
        o = a[:, 0:128] * scale + b_ref[...]
        if elu:
            o = jnp.where(o > 0.0, o, jnp.exp(o) - 1.0)
        o_ref[...] = o[:, :hc_out]

    return pl.pallas_call(
        body,
        grid=(N // bn,),
        in_specs=[pl.BlockSpec((bn, 144), lambda i: (i, 0)),
                  pl.BlockSpec((1, 128), lambda i: (0, 0)),
                  pl.BlockSpec((16, 128), lambda i: (0, 0))],
        out_specs=pl.BlockSpec((bn, hc_out), lambda i: (i, 0)),
        out_shape=jax.ShapeDtypeStruct((N, hc_out), jnp.float32),
    )(acc, bias_pad, expm)


def _pass_a(dst):
    @functools.partial(
        pl.kernel,
        out_type=jax.ShapeDtypeStruct((NW, P * 16), jnp.int32),
        mesh=plsc.VectorSubcoreMesh(**_MESH),
        compiler_params=pltpu.CompilerParams(needs_layout_passes=False, use_tc_tiling_on_sc=False),
        scratch_types=[pltpu.VMEM((1008,), jnp.int32),
                       pltpu.VMEM((P * 16,), jnp.int32)],
    )
    def k(dst_hbm, cnt_hbm, dbuf, cbuf):
        wid = _wid()
        lane = _iota16()
        ones = jnp.ones((16,), jnp.int32)

        def zero(i, _):
            cbuf[pl.ds(_al8(i * 16), 16)] = jnp.zeros((16,), jnp.int32)
            return 0
        lax.fori_loop(0, P, zero, 0)

        def blk(ib, _):
            pltpu.sync_copy(dst_hbm.at[pl.ds(_al8(wid * EPW + ib * KA), KA)],
                            dbuf.at[pl.ds(0, KA)])

            def grp(g, _):
                d = jnp.clip(dbuf[pl.ds(_al8(g * 16), 16)], 0, N_PAD - 1)
                b = (d * MAGIC) >> MSHIFT
                plsc.addupdate_scatter(cbuf, [b * 16 + lane], ones)
                return 0
            lax.fori_loop(0, NGA, grp, 0)
            d = jnp.clip(dbuf[pl.ds(NGA * 16, 16)], 0, N_PAD - 1)
            b = (d * MAGIC) >> MSHIFT
            plsc.addupdate_scatter(cbuf, [b * 16 + lane], ones,
                                   mask=lane < TAIL)
            return 0
        lax.fori_loop(0, NBLK_AB, blk, 0)
        pltpu.sync_copy(cbuf, cnt_hbm.at[wid])

    return k(dst)


def _pass_b(src, dst, curinit, lloffs, fills, padls, nchs, gstarts):
    @functools.partial(
        pl.kernel,
        out_type=[jax.ShapeDtypeStruct((RECSZ,), jnp.int32),
                  jax.ShapeDtypeStruct((RECSZ,), jnp.int32)],
        mesh=plsc.VectorSubcoreMesh(**_MESH),
        compiler_params=pltpu.CompilerParams(needs_layout_passes=False, use_tc_tiling_on_sc=False),
        scratch_types=[pltpu.VMEM((1008,), jnp.int32),
                       pltpu.VMEM((1008,), jnp.int32),
                       pltpu.VMEM((144,), jnp.int32),
                       pltpu.VMEM((16,), jnp.int32),
                       pltpu.VMEM((16,), jnp.int32),
                       pltpu.VMEM((LSZ,), jnp.int32),
                       pltpu.VMEM((LSZ,), jnp.int32),
                       pltpu.VMEM((144,), jnp.int32),
                       pltpu.VMEM((144,), jnp.int32),
                       pltpu.VMEM((144,), jnp.int32),
                       pltpu.VMEM((144,), jnp.int32),
                       pltpu.VMEM((144,), jnp.int32),
                       pltpu.VMEM((16,), jnp.int32),
                       pltpu.SemaphoreType.DMA],
    )
    def k(src_hbm, dst_hbm, cur_hbm, ll_hbm, fi_hbm, pa_hbm, nc_hbm, gs_hbm,
          ra_hbm, rb_hbm,
          sbuf, dbuf, curb, tmp, postmp, lsd, lei, lb, fb, pb, nb, gb,
          dummy, sem):
        wid = _wid()
        lane = _iota16()
        sentv = jnp.full((16,), SENT, jnp.int32)
        zerov = jnp.zeros((16,), jnp.int32)
        pltpu.sync_copy(cur_hbm.at[wid], curb)

        def group(o, base, valid_n):
            s = jnp.clip(sbuf[pl.ds(_al8(o), 16)], 0, N - 1)
            d = jnp.clip(dbuf[pl.ds(_al8(o), 16)], 0, N_PAD - 1)
            b = (d * MAGIC) >> MSHIFT
            if valid_n is not None:
                b = jnp.where(lane < valid_n, b, P)
            sb, lid = plsc.sort_key_val(b, lane)
            tmp[pl.ds(0, 16)] = sb
            prev = plsc.load_gather(tmp, [jnp.maximum(lane - 1, 0)])
            startf = jnp.logical_or(lane == 0, sb != prev)
            run_start = plsc.cummax(jnp.where(startf, lane, 0))
            rank = lane - run_start
            basep = plsc.load_gather(curb, [sb])
            pos_s = basep + rank
            nxt = plsc.load_gather(tmp, [jnp.minimum(lane + 1, 15)])
            endf = jnp.logical_or(lane == 15, sb != nxt)
            plsc.store_scatter(curb, [sb], pos_s + 1, mask=endf)
            plsc.store_scatter(postmp, [lid], pos_s)
            pos = postmp[pl.ds(0, 16)]
            pk = s | ((d - b * NR) << 16)
            if valid_n is not None:
                pk = jnp.where(lane < valid_n, pk, SENT)
            plsc.store_scatter(lsd, [pos], pk)
            plsc.store_scatter(lei, [pos], base + o + lane)

        def blk(ib, _):
            base = wid * EPW + ib * KA
            pltpu.sync_copy(src_hbm.at[pl.ds(_al8(base), KA)], sbuf.at[pl.ds(0, KA)])
            pltpu.sync_copy(dst_hbm.at[pl.ds(_al8(base), KA)], dbuf.at[pl.ds(0, KA)])

            def grp(g, _):
                group(g * 16, base, None)
                return 0
            lax.fori_loop(0, NGA, grp, 0)
            group(NGA * 16, base, TAIL)
            return 0
        lax.fori_loop(0, NBLK_AB, blk, 0)

        pltpu.sync_copy(fi_hbm.at[wid], fb)
        pltpu.sync_copy(pa_hbm.at[wid], pb)
        for bb in range(8):
            fv = fb[pl.ds(bb * 16, 16)]
            pv = pb[pl.ds(bb * 16, 16)]
            for r in range(16):
                idx = fv[r] + lane
                m = lane < pv[r]
                plsc.store_scatter(lsd, [idx], sentv, mask=m)
                plsc.store_scatter(lei, [idx], zerov, mask=m)

        pltpu.sync_copy(ll_hbm.at[wid], lb)
        pltpu.sync_copy(nc_hbm.at[wid], nb)
        pltpu.sync_copy(gs_hbm.at[wid], gb)
        prev_n = None

        def drain(count):
            def dr(i, _):
                pltpu.make_async_copy(
                    ra_hbm.at[pl.ds(0, CH)], dummy, sem).wait()
                return 0
            lax.fori_loop(0, count, dr, 0)

        for bb in range(8):
            lv = lb[pl.ds(bb * 16, 16)]
            nv = nb[pl.ds(bb * 16, 16)]
            gv = gb[pl.ds(bb * 16, 16)]
            for r in range(16):
                l0, n0, g0 = lv[r], nv[r], gv[r]

                def cp(ic, _):
                    pltpu.async_copy(lsd.at[pl.ds(_al8(l0 + ic * CH), CH)],
                                     ra_hbm.at[pl.ds(_al8(g0 + ic * CH), CH)], sem)
                    pltpu.async_copy(lei.at[pl.ds(_al8(l0 + ic * CH), CH)],
                                     rb_hbm.at[pl.ds(_al8(g0 + ic * CH), CH)], sem)
                    return 0
                lax.fori_loop(0, n0, cp, 0)
                if prev_n is not None:
                    drain(2 * prev_n)
                prev_n = n0
        drain(2 * prev_n)

    return k(src, dst, curinit, lloffs, fills, padls, nchs, gstarts)


def _pass_c(heads, off, nch, reca, recb_, aedge, hext, tdstb, bins):
    @functools.partial(
        pl.kernel,
        out_type=jax.ShapeDtypeStruct((N_PAD, 144), jnp.float32),
        mesh=plsc.VectorSubcoreMesh(**_MESH),
        compiler_params=pltpu.CompilerParams(needs_layout_passes=False, use_tc_tiling_on_sc=False),
        scratch_types=[pltpu.VMEM((NR, 144), jnp.float32),
                       pltpu.VMEM((NR, 8), jnp.float32),
                       pltpu.VMEM((16,), jnp.int32),
                       pltpu.VMEM((2, KC), jnp.int32),
                       pltpu.VMEM((2, KC), jnp.int32),
                       pltpu.VMEM((2, KC, 16), jnp.float32),
                       pltpu.VMEM((2, KC, 144), jnp.float32),
                       pltpu.VMEM((2, KC), jnp.int32),
                       pltpu.VMEM((2, KC), jnp.int32),
                       pltpu.VMEM((2, KC), jnp.int32),
                       pltpu.VMEM((KC, 16), jnp.float32),
                       pltpu.SemaphoreType.DMA,
                       pltpu.SemaphoreType.DMA,
                       pltpu.SemaphoreType.DMA,
                       pltpu.SemaphoreType.DMA,
                       pltpu.SemaphoreType.DMA,
                       pltpu.SemaphoreType.DMA],
    )
    def k(ra_hbm, rb_hbm, ae_hbm, hx_hbm, td_hbm, bins_hbm, out_hbm,
          acc, tl, binb, ra2, rb2, aeb2, hbuf2, sidx2, eidx2, dbuf2, wbuf,
          sem_r0, sem_r1, sem_a0, sem_a1, sem_h0, sem_h1):
        wid = _wid()
        lane = _iota16()
        z16 = jnp.zeros((16,), jnp.int32)
        zf = jnp.zeros((16,), jnp.float32)
        mask_h = jnp.where(lane < heads, 1.0, 0.0).astype(jnp.float32)
        sem_r = (sem_r0, sem_r1)
        sem_a = (sem_a0, sem_a1)
        sem_h = (sem_h0, sem_h1)

        def part_body(j, _):
            part = wid * 4 + j
            node_base = part * NR
            pltpu.sync_copy(td_hbm.at[pl.ds(_al8(node_base), NR)], tl)
            pltpu.sync_copy(bins_hbm.at[part], binb)

            def za(i, _):
                for c in range(9):
                    acc[i, pl.ds(c * 16, 16)] = zf
                return 0
            lax.fori_loop(0, NR, za, 0)

            bv = binb[pl.ds(0, 16)]
            e0 = bv[0]
            ne = bv[1]
            nblk = (ne + KC - 1) >> 7

            def fire(s, ib):
                eb = _al8(e0 + ib * KC)
                pltpu.async_copy(ra_hbm.at[pl.ds(eb, KC)], ra2.at[s], sem_r[s])
                pltpu.async_copy(rb_hbm.at[pl.ds(eb, KC)], rb2.at[s], sem_r[s])
                pltpu.make_async_copy(ra_hbm.at[pl.ds(0, KC)], ra2.at[s],
                                      sem_r[s]).wait()
                pltpu.make_async_copy(rb_hbm.at[pl.ds(0, KC)], rb2.at[s],
                                      sem_r[s]).wait()
                for g in range(KC // 16):
                    v = ra2[s, pl.ds(g * 16, 16)]
                    ev = rb2[s, pl.ds(g * 16, 16)]
                    sidx2[s, pl.ds(g * 16, 16)] = jnp.clip(v & 0xFFFF, 0, N - 1)
                    eidx2[s, pl.ds(g * 16, 16)] = jnp.clip(ev, 0, E - 1)
                    dbuf2[s, pl.ds(g * 16, 16)] = (v >> 16) & 0x1FF
                pltpu.async_copy(ae_hbm.at[eidx2.at[s]], aeb2.at[s], sem_a[s])
                pltpu.async_copy(hx_hbm.at[sidx2.at[s]], hbuf2.at[s], sem_h[s])

            def consume(s, ib):
                rem = ne - ib * KC
                pltpu.make_async_copy(ae_hbm.at[pl.ds(0, KC)], aeb2.at[s],
                                      sem_a[s]).wait()
                pltpu.make_async_copy(hx_hbm.at[pl.ds(0, KC)], hbuf2.at[s],
                                      sem_h[s]).wait()
                for g in range(KC // 16):
                    e16 = g * 16 + lane
                    dvr = dbuf2[s, pl.ds(g * 16, 16)]
                    dv = jnp.minimum(dvr, NR - 1)
                    vm = jnp.logical_and(e16 < rem, dvr < NR)
                    for h in range(heads):
                        a_s = plsc.load_gather(hbuf2.at[s],
                                               [e16, z16 + (128 + h)])
                        a_d = plsc.load_gather(tl, [dv, z16 + h])
                        bnd = plsc.load_gather(tl, [dv, z16 + (4 + h)])
                        a_e = plsc.load_gather(aeb2.at[s],
                                               [e16, z16 + (off + h)])
                        al = a_s + a_d + a_e
                        al = jnp.maximum(al, 0.2 * al)
                        w = jnp.exp(al - bnd)
                        w = jnp.where(vm, w, 0.0)
                        plsc.store_scatter(wbuf, [e16, z16 + h], w)

                def acc_grp(g, _):
                    base16 = g * 16
                    dv16 = jnp.minimum(dbuf2[s, pl.ds(_al8(base16), 16)],
                                       NR - 1)
                    for k in range(16):
                        i_row = base16 + k
                        d = dv16[k]
                        wrow = wbuf[i_row, pl.ds(0, 16)]
                        for c in range(nch):
                            hv = hbuf2[s, i_row, pl.ds(c * 16, 16)]
                            plsc.addupdate(acc.at[d, pl.ds(c * 16, 16)],
                                           wrow[c // 2] * hv)
                        plsc.addupdate(acc.at[d, pl.ds(128, 16)],
                                       wrow * mask_h)
                    return 0
                lax.fori_loop(0, KC // 16, acc_grp, 0)

            @pl.when(nblk > 0)
            def _():
                fire(0, 0)

            def pair(ip, _):
                ib0 = 2 * ip
                ib1 = ib0 + 1

                @pl.when(ib1 < nblk)
                def _():
                    fire(1, ib1)
                consume(0, ib0)

                @pl.when(ib1 + 1 < nblk)
                def _():
                    fire(0, ib1 + 1)

                @pl.when(ib1 < nblk)
                def _():
                    consume(1, ib1)
                return 0
            lax.fori_loop(0, (nblk + 1) >> 1, pair, 0)
            pltpu.sync_copy(acc, out_hbm.at[pl.ds(_al8(node_base), NR)])
            return 0
        lax.fori_loop(0, 4, part_body, 0)

    return k(reca, recb_, aedge, hext, tdstb, bins)


def kernel(x, edge_index, edge_attr, params):
    src = edge_index[0].astype(jnp.int32)
    dst = edge_index[1].astype(jnp.int32)
    f32 = jnp.float32
    i32 = jnp.int32

    we_cols, w_pads, msrcs, mdsts, biases, expms, hcs = [], [], [], [], [], [], []
    for p in params:
        heads = p['att_src'].shape[1]
        ch = p['att_src'].shape[2]
        hc = heads * ch
        din = p['W'].shape[0]
        we = p['W_e'].reshape(p['W_e'].shape[0], heads, ch)
        we_cols.append((we * p['att_edge']).sum(-1))
        w_pads.append(jnp.pad(p['W'], ((0, 0), (0, 128 - hc))))
        rows = jnp.arange(hc)
        msrc = jnp.zeros((128, 16), f32).at[rows, rows // ch].set(
            p['att_src'].reshape(hc))
        mdst = jnp.zeros((128, 16), f32).at[rows, rows // ch].set(
            p['att_dst'].reshape(hc))
        msrcs.append(msrc)
        mdsts.append(mdst)
        biases.append(jnp.pad(p['bias'], (0, 128 - p['bias'].shape[0]))
                      .reshape(1, 128))
        expms.append(jnp.zeros((16, 128), f32).at[rows // ch, rows].set(1.0))
        hcs.append(hc)

    we_all = jnp.concatenate(we_cols, axis=1)
    we_all = jnp.pad(we_all, ((0, 0), (0, 16 - we_all.shape[1])))
    aedge_all, emax16 = _k0_aedge(edge_attr, we_all)

    cnt = _pass_a(dst).reshape(NW, P, 16).sum(-1)
    rnd = ((cnt + CH - 1) // CH) * CH
    ne_pad = rnd.sum(0)
    astart = jnp.concatenate(
        [jnp.zeros((1,), i32), jnp.cumsum(ne_pad)])[:P]
    gstart = astart[None, :] + jnp.cumsum(rnd, axis=0) - rnd
    lloff = jnp.cumsum(rnd, axis=1) - rnd

    def _pad144(a, trail=0):
        ext = jnp.full((NW, 144 - P), trail, i32)
        return jnp.concatenate([a.astype(i32), ext], axis=1)

    curinit = _pad144(lloff).at[:, P].set(LTRASH)
    lloffs = _pad144(lloff)
    fills = _pad144(lloff + cnt)
    padls = _pad144(rnd - cnt)
    nchs = _pad144(rnd // CH)
    gstarts = _pad144(gstart)
    bins = jnp.concatenate(
        [astart[:, None], ne_pad[:, None], jnp.zeros((P, 14), i32)], axis=1)
    reca, recb = _pass_b(src, dst, curinit, lloffs, fills, padls, nchs,
                         gstarts)

    h = x
    layer_heads = [p['att_src'].shape[1] for p in params]
    for li, p in enumerate(params):
        heads = layer_heads[li]
        hext, adst16, smax = _k1_dense(h, w_pads[li], msrcs[li], mdsts[li])
        td = _k2_tdstb(adst16, smax[:, :4], emax16[:, OFFS[li]:OFFS[li] + 4])
        acc = _pass_c(heads, OFFS[li], (heads * 32) // 16,
                      reca, recb, aedge_all, hext, td, bins)
        out_w = hcs[li] if li < len(params) - 1 else params[li]['bias'].shape[0]
        h = _k3_norm(acc, biases[li], expms[li], out_w, elu=li < len(params) - 1)
    return h


# speedup vs baseline: 47.0228x; 1.0004x over previous
import functools

import jax
import jax.numpy as jnp
from jax import lax
from jax.experimental import pallas as pl
from jax.experimental.pallas import tpu as pltpu
from jax.experimental.pallas import tpu_sc as plsc

N = 50000
E = 800000
NW = 32
EPW = E // NW
KA = 1000
NGA = KA // 16
TAIL = KA - NGA * 16
NBLK_AB = EPW // KA
NR = 416
P = 128
N_PAD = NR * P
MAGIC, MSHIFT = 20165, 23
CH = 16
RECSZ = E + NW * P * (CH - 1) + 256
LTRASH = EPW + P * (CH - 1) + 24
LSZ = LTRASH + 224
SENT = 511 << 16
KC = 160
OFFS = (0, 4, 8, 12)

_MESH = dict(core_axis_name="c", subcore_axis_name="s")


def _al8(x):
    return pl.multiple_of(x, 8)

def _iota16():
    return lax.broadcasted_iota(jnp.int32, (16,), 0)


def _wid():
    return lax.axis_index("s") * 2 + lax.axis_index("c")


def _k0_aedge(edge_attr, we_all):
    be = 16000

    def body(ea_ref, w_ref, out_ref, mx_ref):
        i = pl.program_id(0)
        r = jnp.dot(ea_ref[...], w_ref[...], preferred_element_type=jnp.float32)
        out_ref[...] = r
        m = jnp.max(r, axis=0, keepdims=True)

        @pl.when(i == 0)
        def _():
            mx_ref[...] = m

        @pl.when(i > 0)
        def _():
            mx_ref[...] = jnp.maximum(mx_ref[...], m)

    return pl.pallas_call(
        body,
        grid=(E // be,),
        in_specs=[pl.BlockSpec((be, 16), lambda i: (i, 0)),
                  pl.BlockSpec((16, 16), lambda i: (0, 0))],
        out_specs=[pl.BlockSpec((be, 16), lambda i: (i, 0)),
                   pl.BlockSpec((1, 16), lambda i: (0, 0))],
        out_shape=[jax.ShapeDtypeStruct((E, 16), jnp.float32),
                   jax.ShapeDtypeStruct((1, 16), jnp.float32)],
    )(edge_attr, we_all)


def _k1_dense(x, w_pad, msrc, mdst):
    bn = 2000
    din = x.shape[1]

    def body(x_ref, w_ref, ms_ref, md_ref, hx_ref, ad_ref, sm_ref):
        i = pl.program_id(0)
        h = jnp.dot(x_ref[...], w_ref[...], preferred_element_type=jnp.float32)
        asrc = jnp.dot(h, ms_ref[...], preferred_element_type=jnp.float32)
        adst = jnp.dot(h, md_ref[...], preferred_element_type=jnp.float32)
        hx_ref[...] = jnp.concatenate([h, asrc], axis=1)
        ad_ref[...] = adst
        m = jnp.max(asrc, axis=0, keepdims=True)

        @pl.when(i == 0)
        def _():
            sm_ref[...] = m

        @pl.when(i > 0)
        def _():
            sm_ref[...] = jnp.maximum(sm_ref[...], m)

    return pl.pallas_call(
        body,
        grid=(N // bn,),
        in_specs=[pl.BlockSpec((bn, din), lambda i: (i, 0)),
                  pl.BlockSpec((din, 128), lambda i: (0, 0)),
                  pl.BlockSpec((128, 16), lambda i: (0, 0)),
                  pl.BlockSpec((128, 16), lambda i: (0, 0))],
        out_specs=[pl.BlockSpec((bn, 144), lambda i: (i, 0)),
                   pl.BlockSpec((bn, 16), lambda i: (i, 0)),
                   pl.BlockSpec((1, 16), lambda i: (0, 0))],
        out_shape=[jax.ShapeDtypeStruct((N, 144), jnp.float32),
                   jax.ShapeDtypeStruct((N_PAD, 16), jnp.float32),
                   jax.ShapeDtypeStruct((1, 16), jnp.float32)],
    )(x, w_pad, msrc, mdst)


def _k2_tdstb(adst16, smax4, emax4):
    bn = 2048

    def body(ad_ref, sm_ref, em_ref, t_ref):
        a4 = ad_ref[...][:, :4]
        z = a4 + sm_ref[...] + em_ref[...]
        b = jnp.maximum(z, 0.2 * z)
        t_ref[...] = jnp.concatenate([a4, b], axis=1)

    return pl.pallas_call(
        body,
        grid=(N_PAD // bn,),
        in_specs=[pl.BlockSpec((bn, 16), lambda i: (i, 0)),
                  pl.BlockSpec((1, 4), lambda i: (0, 0)),
                  pl.BlockSpec((1, 4), lambda i: (0, 0))],
        out_specs=pl.BlockSpec((bn, 8), lambda i: (i, 0)),
        out_shape=jax.ShapeDtypeStruct((N_PAD, 8), jnp.float32),
    )(adst16, smax4, emax4)


def _k3_norm(acc, bias_pad, expm, hc_out, elu):
    bn = 2000

    def body(a_ref, b_ref, e_ref, o_ref):
        a = a_ref[...]
        recip = 1.0 / (a[:, 128:144] + 1e-16)
        scale = jnp.dot(recip, e_ref[...], preferred_element_type=jnp.float32)
        o = a[:, 0:128] * scale + b_ref[...]
        if elu:
            o = jnp.where(o > 0.0, o, jnp.exp(o) - 1.0)
        o_ref[...] = o[:, :hc_out]

    return pl.pallas_call(
        body,
        grid=(N // bn,),
        in_specs=[pl.BlockSpec((bn, 144), lambda i: (i, 0)),
                  pl.BlockSpec((1, 128), lambda i: (0, 0)),
                  pl.BlockSpec((16, 128), lambda i: (0, 0))],
        out_specs=pl.BlockSpec((bn, hc_out), lambda i: (i, 0)),
        out_shape=jax.ShapeDtypeStruct((N, hc_out), jnp.float32),
    )(acc, bias_pad, expm)


def _pass_a(dst):
    @functools.partial(
        pl.kernel,
        out_type=jax.ShapeDtypeStruct((NW, P * 16), jnp.int32),
        mesh=plsc.VectorSubcoreMesh(**_MESH),
        compiler_params=pltpu.CompilerParams(needs_layout_passes=False, use_tc_tiling_on_sc=False),
        scratch_types=[pltpu.VMEM((1008,), jnp.int32),
                       pltpu.VMEM((P * 16,), jnp.int32)],
    )
    def k(dst_hbm, cnt_hbm, dbuf, cbuf):
        wid = _wid()
        lane = _iota16()
        ones = jnp.ones((16,), jnp.int32)

        def zero(i, _):
            cbuf[pl.ds(_al8(i * 16), 16)] = jnp.zeros((16,), jnp.int32)
            return 0
        lax.fori_loop(0, P, zero, 0)

        def blk(ib, _):
            pltpu.sync_copy(dst_hbm.at[pl.ds(_al8(wid * EPW + ib * KA), KA)],
                            dbuf.at[pl.ds(0, KA)])

            def grp(g, _):
                d = jnp.clip(dbuf[pl.ds(_al8(g * 16), 16)], 0, N_PAD - 1)
                b = (d * MAGIC) >> MSHIFT
                plsc.addupdate_scatter(cbuf, [b * 16 + lane], ones)
                return 0
            lax.fori_loop(0, NGA, grp, 0)
            d = jnp.clip(dbuf[pl.ds(NGA * 16, 16)], 0, N_PAD - 1)
            b = (d * MAGIC) >> MSHIFT
            plsc.addupdate_scatter(cbuf, [b * 16 + lane], ones,
                                   mask=lane < TAIL)
            return 0
        lax.fori_loop(0, NBLK_AB, blk, 0)
        pltpu.sync_copy(cbuf, cnt_hbm.at[wid])

    return k(dst)


def _pass_b(src, dst, curinit, lloffs, fills, padls, nchs, gstarts):
    @functools.partial(
        pl.kernel,
        out_type=[jax.ShapeDtypeStruct((RECSZ,), jnp.int32),
                  jax.ShapeDtypeStruct((RECSZ,), jnp.int32)],
        mesh=plsc.VectorSubcoreMesh(**_MESH),
        compiler_params=pltpu.CompilerParams(needs_layout_passes=False, use_tc_tiling_on_sc=False),
        scratch_types=[pltpu.VMEM((1008,), jnp.int32),
                       pltpu.VMEM((1008,), jnp.int32),
                       pltpu.VMEM((144,), jnp.int32),
                       pltpu.VMEM((16,), jnp.int32),
                       pltpu.VMEM((16,), jnp.int32),
                       pltpu.VMEM((LSZ,), jnp.int32),
                       pltpu.VMEM((LSZ,), jnp.int32),
                       pltpu.VMEM((144,), jnp.int32),
                       pltpu.VMEM((144,), jnp.int32),
                       pltpu.VMEM((144,), jnp.int32),
                       pltpu.VMEM((144,), jnp.int32),
                       pltpu.VMEM((144,), jnp.int32),
                       pltpu.VMEM((16,), jnp.int32),
                       pltpu.SemaphoreType.DMA],
    )
    def k(src_hbm, dst_hbm, cur_hbm, ll_hbm, fi_hbm, pa_hbm, nc_hbm, gs_hbm,
          ra_hbm, rb_hbm,
          sbuf, dbuf, curb, tmp, postmp, lsd, lei, lb, fb, pb, nb, gb,
          dummy, sem):
        wid = _wid()
        lane = _iota16()
        sentv = jnp.full((16,), SENT, jnp.int32)
        zerov = jnp.zeros((16,), jnp.int32)
        pltpu.sync_copy(cur_hbm.at[wid], curb)

        def group(o, base, valid_n):
            s = jnp.clip(sbuf[pl.ds(_al8(o), 16)], 0, N - 1)
            d = jnp.clip(dbuf[pl.ds(_al8(o), 16)], 0, N_PAD - 1)
            b = (d * MAGIC) >> MSHIFT
            if valid_n is not None:
                b = jnp.where(lane < valid_n, b, P)
            sb, lid = plsc.sort_key_val(b, lane)
            tmp[pl.ds(0, 16)] = sb
            prev = plsc.load_gather(tmp, [jnp.maximum(lane - 1, 0)])
            startf = jnp.logical_or(lane == 0, sb != prev)
            run_start = plsc.cummax(jnp.where(startf, lane, 0))
            rank = lane - run_start
            basep = plsc.load_gather(curb, [sb])
            pos_s = basep + rank
            nxt = plsc.load_gather(tmp, [jnp.minimum(lane + 1, 15)])
            endf = jnp.logical_or(lane == 15, sb != nxt)
            plsc.store_scatter(curb, [sb], pos_s + 1, mask=endf)
            plsc.store_scatter(postmp, [lid], pos_s)
            pos = postmp[pl.ds(0, 16)]
            pk = s | ((d - b * NR) << 16)
            if valid_n is not None:
                pk = jnp.where(lane < valid_n, pk, SENT)
            plsc.store_scatter(lsd, [pos], pk)
            plsc.store_scatter(lei, [pos], base + o + lane)

        def blk(ib, _):
            base = wid * EPW + ib * KA
            pltpu.sync_copy(src_hbm.at[pl.ds(_al8(base), KA)], sbuf.at[pl.ds(0, KA)])
            pltpu.sync_copy(dst_hbm.at[pl.ds(_al8(base), KA)], dbuf.at[pl.ds(0, KA)])

            def grp(g, _):
                group(g * 16, base, None)
                return 0
            lax.fori_loop(0, NGA, grp, 0)
            group(NGA * 16, base, TAIL)
            return 0
        lax.fori_loop(0, NBLK_AB, blk, 0)

        pltpu.sync_copy(fi_hbm.at[wid], fb)
        pltpu.sync_copy(pa_hbm.at[wid], pb)
        for bb in range(8):
            fv = fb[pl.ds(bb * 16, 16)]
            pv = pb[pl.ds(bb * 16, 16)]
            for r in range(16):
                idx = fv[r] + lane
                m = lane < pv[r]
                plsc.store_scatter(lsd, [idx], sentv, mask=m)
                plsc.store_scatter(lei, [idx], zerov, mask=m)

        pltpu.sync_copy(ll_hbm.at[wid], lb)
        pltpu.sync_copy(nc_hbm.at[wid], nb)
        pltpu.sync_copy(gs_hbm.at[wid], gb)
        prev_n = None

        def drain(count):
            def dr(i, _):
                pltpu.make_async_copy(
                    ra_hbm.at[pl.ds(0, CH)], dummy, sem).wait()
                return 0
            lax.fori_loop(0, count, dr, 0)

        for bb in range(8):
            lv = lb[pl.ds(bb * 16, 16)]
            nv = nb[pl.ds(bb * 16, 16)]
            gv = gb[pl.ds(bb * 16, 16)]
            for r in range(16):
                l0, n0, g0 = lv[r], nv[r], gv[r]

                def cp(ic, _):
                    pltpu.async_copy(lsd.at[pl.ds(_al8(l0 + ic * CH), CH)],
                                     ra_hbm.at[pl.ds(_al8(g0 + ic * CH), CH)], sem)
                    pltpu.async_copy(lei.at[pl.ds(_al8(l0 + ic * CH), CH)],
                                     rb_hbm.at[pl.ds(_al8(g0 + ic * CH), CH)], sem)
                    return 0
                lax.fori_loop(0, n0, cp, 0)
                if prev_n is not None:
                    drain(2 * prev_n)
                prev_n = n0
        drain(2 * prev_n)

    return k(src, dst, curinit, lloffs, fills, padls, nchs, gstarts)


def _pass_c(heads, off, nch, reca, recb_, aedge, hext, tdstb, bins):
    @functools.partial(
        pl.kernel,
        out_type=jax.ShapeDtypeStruct((N_PAD, 144), jnp.float32),
        mesh=plsc.VectorSubcoreMesh(**_MESH),
        compiler_params=pltpu.CompilerParams(needs_layout_passes=False, use_tc_tiling_on_sc=False),
        scratch_types=[pltpu.VMEM((NR, 144), jnp.float32),
                       pltpu.VMEM((NR, 8), jnp.float32),
                       pltpu.VMEM((16,), jnp.int32),
                       pltpu.VMEM((2, KC), jnp.int32),
                       pltpu.VMEM((2, KC), jnp.int32),
                       pltpu.VMEM((2, KC, 16), jnp.float32),
                       pltpu.VMEM((2, KC, 144), jnp.float32),
                       pltpu.VMEM((2, KC), jnp.int32),
                       pltpu.VMEM((2, KC), jnp.int32),
                       pltpu.VMEM((2, KC), jnp.int32),
                       pltpu.VMEM((KC, 16), jnp.float32),
                       pltpu.SemaphoreType.DMA,
                       pltpu.SemaphoreType.DMA,
                       pltpu.SemaphoreType.DMA,
                       pltpu.SemaphoreType.DMA,
                       pltpu.SemaphoreType.DMA,
                       pltpu.SemaphoreType.DMA],
    )
    def k(ra_hbm, rb_hbm, ae_hbm, hx_hbm, td_hbm, bins_hbm, out_hbm,
          acc, tl, binb, ra2, rb2, aeb2, hbuf2, sidx2, eidx2, dbuf2, wbuf,
          sem_r0, sem_r1, sem_a0, sem_a1, sem_h0, sem_h1):
        wid = _wid()
        lane = _iota16()
        z16 = jnp.zeros((16,), jnp.int32)
        zf = jnp.zeros((16,), jnp.float32)
        mask_h = jnp.where(lane < heads, 1.0, 0.0).astype(jnp.float32)
        sem_r = (sem_r0, sem_r1)
        sem_a = (sem_a0, sem_a1)
        sem_h = (sem_h0, sem_h1)

        def part_body(j, _):
            part = wid * 4 + j
            node_base = part * NR
            pltpu.sync_copy(td_hbm.at[pl.ds(_al8(node_base), NR)], tl)
            pltpu.sync_copy(bins_hbm.at[part], binb)

            def za(i, _):
                for c in range(9):
                    acc[i, pl.ds(c * 16, 16)] = zf
                return 0
            lax.fori_loop(0, NR, za, 0)

            bv = binb[pl.ds(0, 16)]
            e0 = bv[0]
            ne = bv[1]
            nblk = (((ne + KC - 1) >> 5) * 52429) >> 18

            def fire(s, ib):
                eb = _al8(e0 + ib * KC)
                pltpu.async_copy(ra_hbm.at[pl.ds(eb, KC)], ra2.at[s], sem_r[s])
                pltpu.async_copy(rb_hbm.at[pl.ds(eb, KC)], rb2.at[s], sem_r[s])
                pltpu.make_async_copy(ra_hbm.at[pl.ds(0, KC)], ra2.at[s],
                                      sem_r[s]).wait()
                pltpu.make_async_copy(rb_hbm.at[pl.ds(0, KC)], rb2.at[s],
                                      sem_r[s]).wait()
                for g in range(KC // 16):
                    v = ra2[s, pl.ds(g * 16, 16)]
                    ev = rb2[s, pl.ds(g * 16, 16)]
                    sidx2[s, pl.ds(g * 16, 16)] = jnp.clip(v & 0xFFFF, 0, N - 1)
                    eidx2[s, pl.ds(g * 16, 16)] = jnp.clip(ev, 0, E - 1)
                    dbuf2[s, pl.ds(g * 16, 16)] = (v >> 16) & 0x1FF
                pltpu.async_copy(ae_hbm.at[eidx2.at[s]], aeb2.at[s], sem_a[s])
                pltpu.async_copy(hx_hbm.at[sidx2.at[s]], hbuf2.at[s], sem_h[s])

            def consume(s, ib):
                rem = ne - ib * KC
                pltpu.make_async_copy(ae_hbm.at[pl.ds(0, KC)], aeb2.at[s],
                                      sem_a[s]).wait()
                pltpu.make_async_copy(hx_hbm.at[pl.ds(0, KC)], hbuf2.at[s],
                                      sem_h[s]).wait()
                for g in range(KC // 16):
                    e16 = g * 16 + lane
                    dvr = dbuf2[s, pl.ds(g * 16, 16)]
                    dv = jnp.minimum(dvr, NR - 1)
                    vm = jnp.logical_and(e16 < rem, dvr < NR)
                    for h in range(heads):
                        a_s = plsc.load_gather(hbuf2.at[s],
                                               [e16, z16 + (128 + h)])
                        a_d = plsc.load_gather(tl, [dv, z16 + h])
                        bnd = plsc.load_gather(tl, [dv, z16 + (4 + h)])
                        a_e = plsc.load_gather(aeb2.at[s],
                                               [e16, z16 + (off + h)])
                        al = a_s + a_d + a_e
                        al = jnp.maximum(al, 0.2 * al)
                        w = jnp.exp(al - bnd)
                        w = jnp.where(vm, w, 0.0)
                        plsc.store_scatter(wbuf, [e16, z16 + h], w)

                def acc_grp(g, _):
                    base16 = g * 16
                    dv16 = jnp.minimum(dbuf2[s, pl.ds(_al8(base16), 16)],
                                       NR - 1)
                    for k in range(16):
                        i_row = base16 + k
                        d = dv16[k]
                        wrow = wbuf[i_row, pl.ds(0, 16)]
                        for c in range(nch):
                            hv = hbuf2[s, i_row, pl.ds(c * 16, 16)]
                            plsc.addupdate(acc.at[d, pl.ds(c * 16, 16)],
                                           wrow[c // 2] * hv)
                        plsc.addupdate(acc.at[d, pl.ds(128, 16)],
                                       wrow * mask_h)
                    return 0
                lax.fori_loop(0, KC // 16, acc_grp, 0)

            @pl.when(nblk > 0)
            def _():
                fire(0, 0)

            def pair(ip, _):
                ib0 = 2 * ip
                ib1 = ib0 + 1

                @pl.when(ib1 < nblk)
                def _():
                    fire(1, ib1)
                consume(0, ib0)

                @pl.when(ib1 + 1 < nblk)
                def _():
                    fire(0, ib1 + 1)

                @pl.when(ib1 < nblk)
                def _():
                    consume(1, ib1)
                return 0
            lax.fori_loop(0, (nblk + 1) >> 1, pair, 0)
            pltpu.sync_copy(acc, out_hbm.at[pl.ds(_al8(node_base), NR)])
            return 0
        lax.fori_loop(0, 4, part_body, 0)

    return k(reca, recb_, aedge, hext, tdstb, bins)


def kernel(x, edge_index, edge_attr, params):
    src = edge_index[0].astype(jnp.int32)
    dst = edge_index[1].astype(jnp.int32)
    f32 = jnp.float32
    i32 = jnp.int32

    we_cols, w_pads, msrcs, mdsts, biases, expms, hcs = [], [], [], [], [], [], []
    for p in params:
        heads = p['att_src'].shape[1]
        ch = p['att_src'].shape[2]
        hc = heads * ch
        din = p['W'].shape[0]
        we = p['W_e'].reshape(p['W_e'].shape[0], heads, ch)
        we_cols.append((we * p['att_edge']).sum(-1))
        w_pads.append(jnp.pad(p['W'], ((0, 0), (0, 128 - hc))))
        rows = jnp.arange(hc)
        msrc = jnp.zeros((128, 16), f32).at[rows, rows // ch].set(
            p['att_src'].reshape(hc))
        mdst = jnp.zeros((128, 16), f32).at[rows, rows // ch].set(
            p['att_dst'].reshape(hc))
        msrcs.append(msrc)
        mdsts.append(mdst)
        biases.append(jnp.pad(p['bias'], (0, 128 - p['bias'].shape[0]))
                      .reshape(1, 128))
        expms.append(jnp.zeros((16, 128), f32).at[rows // ch, rows].set(1.0))
        hcs.append(hc)

    we_all = jnp.concatenate(we_cols, axis=1)
    we_all = jnp.pad(we_all, ((0, 0), (0, 16 - we_all.shape[1])))
    aedge_all, emax16 = _k0_aedge(edge_attr, we_all)

    cnt = _pass_a(dst).reshape(NW, P, 16).sum(-1)
    rnd = ((cnt + CH - 1) // CH) * CH
    ne_pad = rnd.sum(0)
    astart = jnp.concatenate(
        [jnp.zeros((1,), i32), jnp.cumsum(ne_pad)])[:P]
    gstart = astart[None, :] + jnp.cumsum(rnd, axis=0) - rnd
    lloff = jnp.cumsum(rnd, axis=1) - rnd

    def _pad144(a, trail=0):
        ext = jnp.full((NW, 144 - P), trail, i32)
        return jnp.concatenate([a.astype(i32), ext], axis=1)

    curinit = _pad144(lloff).at[:, P].set(LTRASH)
    lloffs = _pad144(lloff)
    fills = _pad144(lloff + cnt)
    padls = _pad144(rnd - cnt)
    nchs = _pad144(rnd // CH)
    gstarts = _pad144(gstart)
    bins = jnp.concatenate(
        [astart[:, None], ne_pad[:, None], jnp.zeros((P, 14), i32)], axis=1)
    reca, recb = _pass_b(src, dst, curinit, lloffs, fills, padls, nchs,
                         gstarts)

    h = x
    layer_heads = [p['att_src'].shape[1] for p in params]
    for li, p in enumerate(params):
        heads = layer_heads[li]
        hext, adst16, smax = _k1_dense(h, w_pads[li], msrcs[li], mdsts[li])
        td = _k2_tdstb(adst16, smax[:, :4], emax16[:, OFFS[li]:OFFS[li] + 4])
        acc = _pass_c(heads, OFFS[li], (heads * 32) // 16,
                      reca, recb, aedge_all, hext, td, bins)
        out_w = hcs[li] if li < len(params) - 1 else params[li]['bias'].shape[0]
        h = _k3_norm(acc, biases[li], expms[li], out_w, elu=li < len(params) - 1)
    return h
